# baseline (speedup 1.0000x reference)
import functools

import jax
import jax.numpy as jnp
from jax.experimental import pallas as pl
from jax.experimental.pallas import tpu as pltpu

C = 10.0


def _decode_kernel(s, cc_ref, odx_ref, ody_ref, mask0_ref, g_ref,
                   initw_ref, WhT_ref, bh_ref, WvT_ref, bv_ref,
                   WqT_ref, bq_ref, WrT_ref, br_ref, V_ref,
                   logp_ref, rew_ref, act_ref,
                   query_ref, mask_ref, inith_ref, last_ref, ref3_ref):
    b, _, e = cc_ref.shape
    h = WqT_ref.shape[1]
    f32 = jnp.float32

    cc = cc_ref[...]

    h_sum = cc[:, 0, :]
    for k in range(1, s):
        h_sum = h_sum + cc[:, k, :]
    h_mean = h_sum / s
    h_bar = jnp.dot(h_mean, WhT_ref[...],
                    preferred_element_type=f32) + bh_ref[...]
    h_rest = jnp.dot(initw_ref[...], WvT_ref[...],
                     preferred_element_type=f32) + bv_ref[...]
    query_ref[...] = h_bar + h_rest
    ref3 = jnp.dot(cc.reshape(b * s, e), WrT_ref[...],
                   preferred_element_type=f32).reshape(b, s, h)
    ref3_ref[...] = ref3 + br_ref[...][None, :, :]

    mask_ref[...] = mask0_ref[...]
    inith_ref[...] = jnp.zeros((b, e), f32)
    last_ref[...] = jnp.zeros((b, 2), f32)
    logp_ref[...] = jnp.zeros((b, 1), f32)
    rew_ref[...] = jnp.zeros((b, 1), f32)
    act_ref[...] = jnp.zeros((b, s), jnp.int32)

    def step(i, _):
        iota_s = jax.lax.broadcasted_iota(jnp.int32, (b, s), 1)
        g = g_ref[i]
        q = jnp.dot(query_ref[...], WqT_ref[...],
                    preferred_element_type=f32) + bq_ref[...]
        t = C * jnp.tanh(q[:, None, :] + ref3_ref[...])
        u_full = jnp.dot(t.reshape(b * s, h), V_ref[...],
                         preferred_element_type=f32).reshape(b, s, s)
        rr = jax.lax.broadcasted_iota(jnp.int32, (b, s, s), 1)
        cs = jax.lax.broadcasted_iota(jnp.int32, (b, s, s), 2)
        u = jnp.sum(jnp.where(rr == cs, u_full, 0.0), axis=1)
        u = jnp.where(mask_ref[...] > 0.0, -1e8, u)
        m = jnp.max(u, axis=1, keepdims=True)
        eu = jnp.exp(u - m)
        p = eu / jnp.sum(eu, axis=1, keepdims=True)
        score = jnp.log(p + 1e-12) + g
        smax = jnp.max(score, axis=1, keepdims=True)
        idx = jnp.min(jnp.where(score == smax, iota_s, s),
                      axis=1, keepdims=True)
        onehot = iota_s == idx
        p_sel = jnp.sum(jnp.where(onehot, p, 0.0), axis=1, keepdims=True)
        logp_ref[...] = logp_ref[...] + jnp.log(p_sel + 1e-12)
        act_ref[...] = jnp.where(iota_s == i, idx, act_ref[...])
        mask_ref[...] = jnp.where(onehot, 1.0, mask_ref[...])
        odx = odx_ref[...]
        ody = ody_ref[...]
        node_x = jnp.sum(jnp.where(onehot, odx, 0.0), axis=1, keepdims=True)
        node_y = jnp.sum(jnp.where(onehot, ody, 0.0), axis=1, keepdims=True)
        last = last_ref[...]
        dx = last[:, 0:1] - node_x
        dy = last[:, 1:2] - node_y
        dist = jnp.sqrt(dx * dx + dy * dy)
        rew_ref[...] = jnp.where(i > 0, rew_ref[...] + dist, rew_ref[...])
        last_ref[...] = jnp.concatenate([node_x, node_y], axis=1)
        idx_e = jnp.broadcast_to(idx, (b, e))
        iota3 = jax.lax.broadcasted_iota(jnp.int32, (b, s, e), 1)
        onehot3 = iota3 == idx_e[:, None, :]
        hcur = jnp.sum(jnp.where(onehot3, cc_ref[...], 0.0), axis=1)
        inith = jnp.where(i == 0, hcur, inith_ref[...])
        inith_ref[...] = inith
        both = jnp.concatenate([inith, hcur], axis=1)
        query_ref[...] = h_bar + (jnp.dot(both, WvT_ref[...],
                                          preferred_element_type=f32)
                                  + bv_ref[...])
        return 0

    jax.lax.fori_loop(0, s, step, 0)


def kernel(cell_context, original_data, high_mask, init_w, Wh, bh, Wv, bv,
           Wq, bq, Wr, br, V):
    b, s, e = cell_context.shape
    h = Wq.shape[0]
    f32 = jnp.float32

    key = jax.random.key(42)
    keys = jax.vmap(jax.random.fold_in, in_axes=(None, 0))(key, jnp.arange(s))
    g = jax.vmap(lambda k: jax.random.gumbel(k, (b, s), f32))(keys)

    odx = original_data[:, :, 0]
    ody = original_data[:, :, 1]
    mask0 = high_mask.astype(f32)

    nc = 2
    bb = b // nc
    out_shape = (
        jax.ShapeDtypeStruct((b, 1), f32),
        jax.ShapeDtypeStruct((b, 1), f32),
        jax.ShapeDtypeStruct((b, s), jnp.int32),
    )
    scratch_shapes = [
        pltpu.VMEM((bb, e), f32),
        pltpu.VMEM((bb, s), f32),
        pltpu.VMEM((bb, e), f32),
        pltpu.VMEM((bb, 2), f32),
        pltpu.VMEM((bb, s, h), f32),
    ]
    rep = lambda shape: pl.BlockSpec(shape, lambda i: (0,) * len(shape))
    in_specs = [
        pl.BlockSpec((bb, s, e), lambda i: (i, 0, 0)),
        pl.BlockSpec((bb, s), lambda i: (i, 0)),
        pl.BlockSpec((bb, s), lambda i: (i, 0)),
        pl.BlockSpec((bb, s), lambda i: (i, 0)),
        pl.BlockSpec((s, bb, s), lambda i: (0, i, 0)),
        rep((1, 2 * e)), rep((e, e)), rep((1, e)), rep((2 * e, e)),
        rep((1, e)), rep((e, h)), rep((1, h)), rep((e, h)), rep((1, h)),
        rep((h, s)),
    ]
    out_specs = (
        pl.BlockSpec((bb, 1), lambda i: (i, 0)),
        pl.BlockSpec((bb, 1), lambda i: (i, 0)),
        pl.BlockSpec((bb, s), lambda i: (i, 0)),
    )
    logp, rew, acts = pl.pallas_call(
        functools.partial(_decode_kernel, s),
        grid=(nc,),
        in_specs=in_specs,
        out_specs=out_specs,
        out_shape=out_shape,
        scratch_shapes=scratch_shapes,
        compiler_params=pltpu.CompilerParams(
            dimension_semantics=("parallel",)),
    )(cell_context, odx, ody, mask0, g,
      init_w.reshape(1, 2 * e), Wh.T, bh.reshape(1, e), Wv.T,
      bv.reshape(1, e), Wq.T, bq.reshape(1, h), Wr.T, br.reshape(1, h),
      jnp.broadcast_to(V.reshape(h, 1), (h, s)))
    return logp.reshape(b), rew.reshape(b), acts


# speedup vs baseline: 2.1661x; 2.1661x over previous
import functools

import jax
import jax.numpy as jnp
from jax.experimental import pallas as pl
from jax.experimental.pallas import tpu as pltpu

C = 10.0


def _decode_kernel(s, ccT_ref, odx_ref, ody_ref, mask0_ref, g_ref,
                   initw_ref, WhT_ref, bh_ref, WvT_ref, bv_ref,
                   WqT_ref, bq_ref, WrT_ref, br_ref, Vm_ref,
                   logp_ref, rew_ref, act_ref,
                   query_ref, mask_ref, inith_ref, last_ref, ref3_ref):
    _, b, e = ccT_ref.shape
    h = WqT_ref.shape[1]
    f32 = jnp.float32

    ccT = ccT_ref[...]

    h_sum = ccT[0]
    for k in range(1, s):
        h_sum = h_sum + ccT[k]
    h_mean = h_sum / s
    h_bar = jnp.dot(h_mean, WhT_ref[...],
                    preferred_element_type=f32) + bh_ref[...]
    h_rest = jnp.dot(initw_ref[...], WvT_ref[...],
                     preferred_element_type=f32) + bv_ref[...]
    query_ref[...] = h_bar + h_rest
    ref3 = jnp.dot(ccT.reshape(s * b, e), WrT_ref[...],
                   preferred_element_type=f32).reshape(s, b, h)
    ref3_ref[...] = ref3 + br_ref[...][None, :, :]

    mask_ref[...] = mask0_ref[...]
    inith_ref[...] = jnp.zeros((b, e), f32)
    last_ref[...] = jnp.zeros((b, 2), f32)
    logp_ref[...] = jnp.zeros((b, 1), f32)
    rew_ref[...] = jnp.zeros((b, 1), f32)
    act_ref[...] = jnp.zeros((b, s), jnp.int32)

    def step(i, _):
        iota_s = jax.lax.broadcasted_iota(jnp.int32, (b, s), 1)
        g = g_ref[i]
        q = jnp.dot(query_ref[...], WqT_ref[...],
                    preferred_element_type=f32) + bq_ref[...]
        t = C * jnp.tanh(q[None, :, :] + ref3_ref[...])
        u_full = jnp.dot(t.reshape(s * b, h), Vm_ref[...],
                         preferred_element_type=f32).reshape(s, b, s)
        rr = jax.lax.broadcasted_iota(jnp.int32, (s, b, s), 0)
        cs = jax.lax.broadcasted_iota(jnp.int32, (s, b, s), 2)
        u = jnp.sum(jnp.where(rr == cs, u_full, 0.0), axis=0)
        u = jnp.where(mask_ref[...] > 0.0, -1e8, u)
        m = jnp.max(u, axis=1, keepdims=True)
        eu = jnp.exp(u - m)
        p = eu / jnp.sum(eu, axis=1, keepdims=True)
        score = jnp.log(p + 1e-12) + g
        smax = jnp.max(score, axis=1, keepdims=True)
        idx = jnp.min(jnp.where(score == smax, iota_s, s),
                      axis=1, keepdims=True)
        onehot = iota_s == idx
        p_sel = jnp.sum(jnp.where(onehot, p, 0.0), axis=1, keepdims=True)
        logp_ref[...] = logp_ref[...] + jnp.log(p_sel + 1e-12)
        act_ref[...] = jnp.where(iota_s == i, idx, act_ref[...])
        mask_ref[...] = jnp.where(onehot, 1.0, mask_ref[...])
        odx = odx_ref[...]
        ody = ody_ref[...]
        node_x = jnp.sum(jnp.where(onehot, odx, 0.0), axis=1, keepdims=True)
        node_y = jnp.sum(jnp.where(onehot, ody, 0.0), axis=1, keepdims=True)
        last = last_ref[...]
        dx = last[:, 0:1] - node_x
        dy = last[:, 1:2] - node_y
        dist = jnp.sqrt(dx * dx + dy * dy)
        rew_ref[...] = jnp.where(i > 0, rew_ref[...] + dist, rew_ref[...])
        last_ref[...] = jnp.concatenate([node_x, node_y], axis=1)
        idx_e = jnp.broadcast_to(idx, (b, e))
        sidx = jax.lax.broadcasted_iota(jnp.int32, (s, b, e), 0)
        onehot3 = sidx == idx_e[None, :, :]
        hcur = jnp.sum(jnp.where(onehot3, ccT_ref[...], 0.0), axis=0)
        inith = jnp.where(i == 0, hcur, inith_ref[...])
        inith_ref[...] = inith
        both = jnp.concatenate([inith, hcur], axis=1)
        query_ref[...] = h_bar + (jnp.dot(both, WvT_ref[...],
                                          preferred_element_type=f32)
                                  + bv_ref[...])
        return 0

    jax.lax.fori_loop(0, s, step, 0)


def kernel(cell_context, original_data, high_mask, init_w, Wh, bh, Wv, bv,
           Wq, bq, Wr, br, V):
    b, s, e = cell_context.shape
    h = Wq.shape[0]
    f32 = jnp.float32

    key = jax.random.key(42)
    keys = jax.vmap(jax.random.fold_in, in_axes=(None, 0))(key, jnp.arange(s))
    g = jax.vmap(lambda k: jax.random.gumbel(k, (b, s), f32))(keys)

    ccT = cell_context.transpose(1, 0, 2)
    odx = original_data[:, :, 0]
    ody = original_data[:, :, 1]
    mask0 = high_mask.astype(f32)

    out_shape = (
        jax.ShapeDtypeStruct((b, 1), f32),
        jax.ShapeDtypeStruct((b, 1), f32),
        jax.ShapeDtypeStruct((b, s), jnp.int32),
    )
    scratch_shapes = [
        pltpu.VMEM((b, e), f32),
        pltpu.VMEM((b, s), f32),
        pltpu.VMEM((b, e), f32),
        pltpu.VMEM((b, 2), f32),
        pltpu.VMEM((s, b, h), f32),
    ]
    logp, rew, acts = pl.pallas_call(
        functools.partial(_decode_kernel, s),
        out_shape=out_shape,
        scratch_shapes=scratch_shapes,
    )(ccT, odx, ody, mask0, g,
      init_w.reshape(1, 2 * e), Wh.T, bh.reshape(1, e), Wv.T,
      bv.reshape(1, e), Wq.T, bq.reshape(1, h), Wr.T, br.reshape(1, h),
      jnp.broadcast_to(V.reshape(h, 1), (h, s)))
    return logp.reshape(b), rew.reshape(b), acts
